# baseline (speedup 1.0000x reference)
import functools

import jax
import jax.numpy as jnp
from jax import lax
from jax.experimental import pallas as pl
from jax.experimental.pallas import tpu as pltpu
from jax.experimental.pallas import tpu_sc as plsc

_T, _H, _E, _F = 128, 1024, 64, 1024
_NC, _NS, _L = 2, 16, 16
_NW = _T // _L
_P = 4


def _logits_body(gate_ref, x_ref, out_ref):
    out_ref[...] = lax.dot_general(
        gate_ref[...], x_ref[...],
        (((0,), (1,)), ((), ())),
        preferred_element_type=jnp.float32)


def _router_body(logitsT_hbm, i1_hbm, i2_hbm, w1_hbm, w2_hbm,
                 logits_v, i1_v, i2_v, w1_v, w2_v,
                 i1_sh, i2_sh, w1_sh, w2_sh, sem):
    cid = lax.axis_index("c")
    sid = lax.axis_index("s")
    active = jnp.logical_and(cid == 0, sid < _NW)
    base = jnp.minimum(sid, _NW - 1) * _L

    @pl.when(active)
    def _work():
        pltpu.sync_copy(logitsT_hbm, logits_v)
        vs = [logits_v[e, pl.ds(base, _L)] for e in range(_E)]
        m1 = vs[0]
        for e in range(1, _E):
            m1 = jnp.maximum(m1, vs[e])
        big = jnp.full((_L,), _E, jnp.int32)
        i1 = big
        for e in range(_E):
            i1 = jnp.minimum(i1, jnp.where(vs[e] == m1, e, _E))
        neg = jnp.full((_L,), -jnp.inf, jnp.float32)
        v2s = [jnp.where(i1 == e, neg, vs[e]) for e in range(_E)]
        m2 = v2s[0]
        for e in range(1, _E):
            m2 = jnp.maximum(m2, v2s[e])
        i2 = big
        for e in range(_E):
            i2 = jnp.minimum(i2, jnp.where(v2s[e] == m2, e, _E))
        r = jnp.exp(m2 - m1)
        w1 = 1.0 / (1.0 + r)
        i1_v[...] = i1
        i2_v[...] = i2
        w1_v[...] = w1
        w2_v[...] = 1.0 - w1
        pltpu.sync_copy(i1_v, i1_sh.at[pl.ds(base, _L)])
        pltpu.sync_copy(i2_v, i2_sh.at[pl.ds(base, _L)])
        pltpu.sync_copy(w1_v, w1_sh.at[pl.ds(base, _L)])
        pltpu.sync_copy(w2_v, w2_sh.at[pl.ds(base, _L)])

    plsc.subcore_barrier()

    @pl.when(jnp.logical_and(cid == 0, sid == 0))
    def _drain():
        pltpu.sync_copy(i1_sh, i1_hbm)
        pltpu.sync_copy(i2_sh, i2_hbm)
        pltpu.sync_copy(w1_sh, w1_hbm)
        pltpu.sync_copy(w2_sh, w2_hbm)


def _prefix_body(x_ref, fc1_ref, fc2_ref, y_ref):
    h = jnp.dot(x_ref[...], fc1_ref[0], preferred_element_type=jnp.float32)
    h = h * jax.nn.sigmoid(h)
    y_ref[0] = jnp.dot(h, fc2_ref[0], preferred_element_type=jnp.float32)


def _ffn_body(x_ref, i1_ref, i2_ref, w1_ref, w2_ref, fc1_ref, fc2_ref,
              yp_ref, out_ref):
    i = pl.program_id(0)
    e = jnp.where(i < _E - _P, i + _P, i - (_E - _P))
    c = (jnp.where(i1_ref[...] == e, w1_ref[...], 0.0)
         + jnp.where(i2_ref[...] == e, w2_ref[...], 0.0))

    @pl.when(i < _E - _P)
    def _compute():
        h = jnp.dot(x_ref[...], fc1_ref[0],
                    preferred_element_type=jnp.float32)
        h = h * jax.nn.sigmoid(h) * c
        y = jnp.dot(h, fc2_ref[0], preferred_element_type=jnp.float32)

        @pl.when(i == 0)
        def _init():
            out_ref[...] = y

        @pl.when(i > 0)
        def _acc():
            out_ref[...] += y

    @pl.when(i >= _E - _P)
    def _fold_staged():
        out_ref[...] += yp_ref[0] * c


def kernel(hidden_states, gate_w, fc1_w, fc2_w):
    logitsT = pl.pallas_call(
        _logits_body,
        out_shape=jax.ShapeDtypeStruct((_E, _T), jnp.float32),
    )(gate_w, hidden_states)

    router = functools.partial(
        pl.kernel,
        mesh=plsc.VectorSubcoreMesh(core_axis_name="c", subcore_axis_name="s"),
        out_type=(
            jax.ShapeDtypeStruct((_T,), jnp.int32),
            jax.ShapeDtypeStruct((_T,), jnp.int32),
            jax.ShapeDtypeStruct((_T,), jnp.float32),
            jax.ShapeDtypeStruct((_T,), jnp.float32),
        ),
        scratch_types=[
            pltpu.VMEM((_E, _T), jnp.float32),
            pltpu.VMEM((_L,), jnp.int32),
            pltpu.VMEM((_L,), jnp.int32),
            pltpu.VMEM((_L,), jnp.float32),
            pltpu.VMEM((_L,), jnp.float32),
            pltpu.VMEM_SHARED((_T,), jnp.int32),
            pltpu.VMEM_SHARED((_T,), jnp.int32),
            pltpu.VMEM_SHARED((_T,), jnp.float32),
            pltpu.VMEM_SHARED((_T,), jnp.float32),
            pltpu.SemaphoreType.DMA,
        ],
    )(_router_body)
    i1, i2, w1, w2 = router(logitsT)

    y_prefix = pl.pallas_call(
        _prefix_body,
        grid=(_P,),
        in_specs=[
            pl.BlockSpec((_T, _H), lambda i: (0, 0)),
            pl.BlockSpec((1, _H, _F), lambda i: (i, 0, 0)),
            pl.BlockSpec((1, _F, _H), lambda i: (i, 0, 0)),
        ],
        out_specs=pl.BlockSpec((1, _T, _H), lambda i: (i, 0, 0)),
        out_shape=jax.ShapeDtypeStruct((_P, _T, _H), jnp.float32),
        compiler_params=pltpu.CompilerParams(
            dimension_semantics=("arbitrary",),
        ),
    )(hidden_states, fc1_w, fc2_w)

    return pl.pallas_call(
        _ffn_body,
        grid=(_E,),
        in_specs=[
            pl.BlockSpec((_T, _H), lambda i: (0, 0)),
            pl.BlockSpec((_T, 1), lambda i: (0, 0)),
            pl.BlockSpec((_T, 1), lambda i: (0, 0)),
            pl.BlockSpec((_T, 1), lambda i: (0, 0)),
            pl.BlockSpec((_T, 1), lambda i: (0, 0)),
            pl.BlockSpec((1, _H, _F), lambda i: (jnp.minimum(i + _P, _E - 1), 0, 0)),
            pl.BlockSpec((1, _F, _H), lambda i: (jnp.minimum(i + _P, _E - 1), 0, 0)),
            pl.BlockSpec((1, _T, _H), lambda i: (jnp.maximum(i - (_E - _P), 0), 0, 0)),
        ],
        out_specs=pl.BlockSpec((_T, _H), lambda i: (0, 0)),
        out_shape=jax.ShapeDtypeStruct((_T, _H), jnp.float32),
        compiler_params=pltpu.CompilerParams(
            dimension_semantics=("arbitrary",),
        ),
    )(hidden_states, i1.reshape(_T, 1), i2.reshape(_T, 1),
      w1.reshape(_T, 1), w2.reshape(_T, 1), fc1_w, fc2_w, y_prefix)


# speedup vs baseline: 1.0143x; 1.0143x over previous
import functools

import jax
import jax.numpy as jnp
from jax import lax
from jax.experimental import pallas as pl
from jax.experimental.pallas import tpu as pltpu
from jax.experimental.pallas import tpu_sc as plsc

_T, _H, _E, _F = 128, 1024, 64, 1024
_NC, _NS, _L = 2, 16, 16
_NW = _T // _L
_P = 4
_RC = 16


def _logits_body(gate_ref, x_ref, out_ref):
    out_ref[...] = lax.dot_general(
        gate_ref[...], x_ref[...],
        (((0,), (1,)), ((), ())),
        preferred_element_type=jnp.float32)


def _router_body(logitsT_hbm, route_hbm, logits_v, route_v, route_sh, sem):
    cid = lax.axis_index("c")
    sid = lax.axis_index("s")
    active = jnp.logical_and(cid == 0, sid < _NW)
    base = jnp.minimum(sid, _NW - 1) * _L

    @pl.when(active)
    def _work():
        pltpu.sync_copy(logitsT_hbm, logits_v)
        vs = [logits_v[e, pl.ds(base, _L)] for e in range(_E)]
        m1 = vs[0]
        for e in range(1, _E):
            m1 = jnp.maximum(m1, vs[e])
        big = jnp.full((_L,), _E, jnp.int32)
        i1 = big
        for e in range(_E):
            i1 = jnp.minimum(i1, jnp.where(vs[e] == m1, e, _E))
        neg = jnp.full((_L,), -jnp.inf, jnp.float32)
        v2s = [jnp.where(i1 == e, neg, vs[e]) for e in range(_E)]
        m2 = v2s[0]
        for e in range(1, _E):
            m2 = jnp.maximum(m2, v2s[e])
        i2 = big
        for e in range(_E):
            i2 = jnp.minimum(i2, jnp.where(v2s[e] == m2, e, _E))
        r = jnp.exp(m2 - m1)
        w1 = 1.0 / (1.0 + r)
        w2 = 1.0 - w1
        i1f = i1.astype(jnp.float32)
        i2f = i2.astype(jnp.float32)
        lane = lax.iota(jnp.int32, _L)
        for tt in range(_L):
            row = jnp.where(
                lane == 0, i1f[tt],
                jnp.where(lane == 1, i2f[tt],
                          jnp.where(lane == 2, w1[tt],
                                    jnp.where(lane == 3, w2[tt], 0.0))))
            route_v[tt, :] = row
        pltpu.sync_copy(route_v, route_sh.at[pl.ds(base, _L), :])

    plsc.subcore_barrier()

    @pl.when(jnp.logical_and(cid == 0, sid == 0))
    def _drain():
        pltpu.sync_copy(route_sh, route_hbm)


def _prefix_body(x_ref, fc1_ref, fc2_ref, y_ref):
    h = jnp.dot(x_ref[...], fc1_ref[0], preferred_element_type=jnp.float32)
    h = h * jax.nn.sigmoid(h)
    y_ref[0] = jnp.dot(h, fc2_ref[0], preferred_element_type=jnp.float32)


def _ffn_body(x_ref, route_ref, fc1_ref, fc2_ref, yp_ref, out_ref):
    i = pl.program_id(0)
    e = jnp.where(i < _E - _P, i + _P, i - (_E - _P))
    ef = e.astype(jnp.float32)
    c = (jnp.where(route_ref[:, 0:1] == ef, route_ref[:, 2:3], 0.0)
         + jnp.where(route_ref[:, 1:2] == ef, route_ref[:, 3:4], 0.0))

    @pl.when(i < _E - _P)
    def _compute():
        h = jnp.dot(x_ref[...], fc1_ref[0],
                    preferred_element_type=jnp.float32)
        h = h * jax.nn.sigmoid(h) * c
        y = jnp.dot(h, fc2_ref[0], preferred_element_type=jnp.float32)

        @pl.when(i == 0)
        def _init():
            out_ref[...] = y

        @pl.when(i > 0)
        def _acc():
            out_ref[...] += y

    @pl.when(i >= _E - _P)
    def _fold_staged():
        out_ref[...] += yp_ref[0] * c


def kernel(hidden_states, gate_w, fc1_w, fc2_w):
    logitsT = pl.pallas_call(
        _logits_body,
        out_shape=jax.ShapeDtypeStruct((_E, _T), jnp.float32),
    )(gate_w, hidden_states)

    router = functools.partial(
        pl.kernel,
        mesh=plsc.VectorSubcoreMesh(core_axis_name="c", subcore_axis_name="s"),
        out_type=jax.ShapeDtypeStruct((_T, _RC), jnp.float32),
        scratch_types=[
            pltpu.VMEM((_E, _T), jnp.float32),
            pltpu.VMEM((_L, _RC), jnp.float32),
            pltpu.VMEM_SHARED((_T, _RC), jnp.float32),
            pltpu.SemaphoreType.DMA,
        ],
    )(_router_body)
    route = router(logitsT)

    y_prefix = pl.pallas_call(
        _prefix_body,
        grid=(_P,),
        in_specs=[
            pl.BlockSpec((_T, _H), lambda i: (0, 0)),
            pl.BlockSpec((1, _H, _F), lambda i: (i, 0, 0)),
            pl.BlockSpec((1, _F, _H), lambda i: (i, 0, 0)),
        ],
        out_specs=pl.BlockSpec((1, _T, _H), lambda i: (i, 0, 0)),
        out_shape=jax.ShapeDtypeStruct((_P, _T, _H), jnp.float32),
        compiler_params=pltpu.CompilerParams(
            dimension_semantics=("arbitrary",),
        ),
    )(hidden_states, fc1_w, fc2_w)

    return pl.pallas_call(
        _ffn_body,
        grid=(_E,),
        in_specs=[
            pl.BlockSpec((_T, _H), lambda i: (0, 0)),
            pl.BlockSpec((_T, _RC), lambda i: (0, 0)),
            pl.BlockSpec((1, _H, _F), lambda i: (jnp.minimum(i + _P, _E - 1), 0, 0)),
            pl.BlockSpec((1, _F, _H), lambda i: (jnp.minimum(i + _P, _E - 1), 0, 0)),
            pl.BlockSpec((1, _T, _H), lambda i: (jnp.maximum(i - (_E - _P), 0), 0, 0)),
        ],
        out_specs=pl.BlockSpec((_T, _H), lambda i: (0, 0)),
        out_shape=jax.ShapeDtypeStruct((_T, _H), jnp.float32),
        compiler_params=pltpu.CompilerParams(
            dimension_semantics=("arbitrary",),
        ),
    )(hidden_states, route, fc1_w, fc2_w, y_prefix)
